# baseline (speedup 1.0000x reference)
import functools

import jax
import jax.numpy as jnp
from jax import lax
from jax.experimental import pallas as pl
from jax.experimental.pallas import tpu as pltpu
from jax.experimental.pallas import tpu_sc as plsc

_ALPHA = 0.1
_NG = 64
_N = 10000
_H = 128
_CH = 128
_NCHUNK = 160
_GRP = 16
_NGRP = _NCHUNK // _GRP
_EPT = _CH * _NCHUNK
_NSHARD = 32
_NPAD = 10240
_RPT = _NPAD // 16


def _scale_chunk(buf, normv, i):
    def _escale(eb, c2):
        e0 = eb * 16
        n16 = normv[i, pl.ds(e0, 16)]
        for j in range(16):
            sc = n16[j]
            for k in range(_H // 16):
                sl = pl.ds(k * 16, 16)
                buf[e0 + j, sl] = buf[e0 + j, sl] * sc
        return c2
    lax.fori_loop(0, _CH // 16, _escale, 0)


def _spmm_kernel(h_hbm, row_hbm, col_hbm, norm_hbm, out_hbm,
                 rowv, colv, normv, bufa, bufb, acc, sema, semb, ssa, ssb):
    c = lax.axis_index("c")
    s = lax.axis_index("s")
    wid = c * 16 + s
    def _zrow(i, carry):
        for j in range(_H // 16):
            bufa[i, pl.ds(j * 16, 16)] = jnp.zeros((16,), jnp.float32)
        return carry
    lax.fori_loop(0, _CH, _zrow, 0)
    base = s * _RPT
    for t in range(_RPT // _CH):
        pltpu.sync_copy(bufa, acc.at[pl.ds(base + t * _CH, _CH)])
    plsc.subcore_barrier()

    def _group(g, carry):
        pltpu.sync_copy(row_hbm.at[wid, pl.ds(g * _GRP, _GRP)], rowv)
        pltpu.sync_copy(col_hbm.at[wid, pl.ds(g * _GRP, _GRP)], colv)
        pltpu.sync_copy(norm_hbm.at[wid, pl.ds(g * _GRP, _GRP)], normv)
        pltpu.async_copy(h_hbm.at[rowv.at[0]], bufa, sema)
        pltpu.async_copy(h_hbm.at[rowv.at[1]], bufb, semb)

        def _pair(k, c1):
            i0 = 2 * k
            i1 = i0 + 1
            pltpu.make_async_copy(h_hbm.at[rowv.at[i0]], bufa, sema).wait()
            _scale_chunk(bufa, normv, i0)
            pltpu.async_copy(bufa, acc.at[colv.at[i0]], ssa, add=True)
            pltpu.make_async_copy(h_hbm.at[rowv.at[i1]], bufb, semb).wait()
            _scale_chunk(bufb, normv, i1)
            pltpu.async_copy(bufb, acc.at[colv.at[i1]], ssb, add=True)
            pltpu.make_async_copy(bufa, acc.at[colv.at[i0]], ssa).wait()
            @pl.when(k < _GRP // 2 - 1)
            def _():
                pltpu.async_copy(h_hbm.at[rowv.at[i0 + 2]], bufa, sema)
            pltpu.make_async_copy(bufb, acc.at[colv.at[i1]], ssb).wait()
            @pl.when(k < _GRP // 2 - 1)
            def _():
                pltpu.async_copy(h_hbm.at[rowv.at[i1 + 2]], bufb, semb)
            return c1
        lax.fori_loop(0, _GRP // 2, _pair, 0)
        return carry
    lax.fori_loop(0, _NGRP, _group, 0)
    plsc.subcore_barrier()
    for t in range(_RPT // _CH):
        pltpu.sync_copy(acc.at[pl.ds(base + t * _CH, _CH)],
                        out_hbm.at[c, pl.ds(base + t * _CH, _CH)])


def _spmm(h, row_p, col_p, norm_p):
    mesh = plsc.VectorSubcoreMesh(core_axis_name="c", subcore_axis_name="s")
    f = pl.kernel(
        _spmm_kernel,
        out_type=jax.ShapeDtypeStruct((2, _NPAD, _H), jnp.float32),
        mesh=mesh,
        scratch_types=[
            pltpu.VMEM((_GRP, _CH), jnp.int32),
            pltpu.VMEM((_GRP, _CH), jnp.int32),
            pltpu.VMEM((_GRP, _CH), jnp.float32),
            pltpu.VMEM((_CH, _H), jnp.float32),
            pltpu.VMEM((_CH, _H), jnp.float32),
            pltpu.VMEM_SHARED((_NPAD, _H), jnp.float32),
            pltpu.SemaphoreType.DMA,
            pltpu.SemaphoreType.DMA,
            pltpu.SemaphoreType.DMA,
            pltpu.SemaphoreType.DMA,
        ],
        compiler_params=pltpu.CompilerParams(needs_layout_passes=False),
    )
    return f(h, row_p, col_p, norm_p)


def _norm_kernel(row_hbm, col_hbm, ew_hbm, norm_hbm, dis_hbm,
                 rowv, colv, ewv, normb, zbuf, disv, degacc, gsem):
    c = lax.axis_index("c")
    s = lax.axis_index("s")
    wid = c * 16 + s
    def _z(i, carry):
        zbuf[pl.ds(i * 16, 16)] = jnp.zeros((16,), jnp.float32)
        return carry
    lax.fori_loop(0, _RPT // 16, _z, 0)
    pltpu.sync_copy(zbuf, degacc.at[pl.ds(s * _RPT, _RPT)])
    plsc.subcore_barrier()

    def _dgroup(g, carry):
        sh = s + 16 * (g % 2)
        gg = g // 2
        pltpu.sync_copy(col_hbm.at[sh, pl.ds(gg * _GRP, _GRP)], colv)
        pltpu.sync_copy(ew_hbm.at[sh, pl.ds(gg * _GRP, _GRP)], ewv)
        def _dchunk(i, c1):
            pltpu.sync_copy(ewv.at[i], degacc.at[colv.at[i]], add=True)
            return c1
        lax.fori_loop(0, _GRP, _dchunk, 0)
        return carry
    lax.fori_loop(0, 2 * _NGRP, _dgroup, 0)
    plsc.subcore_barrier()

    pltpu.sync_copy(degacc, disv)
    def _rsq(k, carry):
        sl = pl.ds(k * 16, 16)
        xv = disv[sl] + 1.0
        iv = 0x5F3759DF - lax.shift_right_logical(plsc.bitcast(xv, jnp.int32), 1)
        y = plsc.bitcast(iv, jnp.float32)
        xh = 0.5 * xv
        for _ in range(3):
            y = y * (1.5 - xh * y * y)
        disv[sl] = y
        return carry
    lax.fori_loop(0, _NPAD // 16, _rsq, 0)

    def _ngroup(g, carry):
        pltpu.sync_copy(row_hbm.at[wid, pl.ds(g * _GRP, _GRP)], rowv)
        pltpu.sync_copy(col_hbm.at[wid, pl.ds(g * _GRP, _GRP)], colv)
        pltpu.sync_copy(ew_hbm.at[wid, pl.ds(g * _GRP, _GRP)], ewv)
        def _nchunk(i, c1):
            for eb in range(_CH // 16):
                sl = pl.ds(eb * 16, 16)
                r16 = rowv[i, sl]
                c16 = colv[i, sl]
                e16 = ewv[i, sl]
                dr = plsc.load_gather(disv, [r16])
                dc = plsc.load_gather(disv, [c16])
                normb[i, sl] = dr * e16 * dc
            return c1
        lax.fori_loop(0, _GRP, _nchunk, 0)
        pltpu.sync_copy(normb, norm_hbm.at[wid, pl.ds(g * _GRP, _GRP)])
        return carry
    lax.fori_loop(0, _NGRP, _ngroup, 0)
    @pl.when(jnp.logical_and(c == 0, s == 0))
    def _():
        pltpu.sync_copy(disv, dis_hbm)


def _norm_sc(row_p, col_p, ew_p):
    mesh = plsc.VectorSubcoreMesh(core_axis_name="c", subcore_axis_name="s")
    f = pl.kernel(
        _norm_kernel,
        out_type=(jax.ShapeDtypeStruct((_NSHARD, _NCHUNK, _CH), jnp.float32),
                  jax.ShapeDtypeStruct((_NPAD,), jnp.float32)),
        mesh=mesh,
        scratch_types=[
            pltpu.VMEM((_GRP, _CH), jnp.int32),
            pltpu.VMEM((_GRP, _CH), jnp.int32),
            pltpu.VMEM((_GRP, _CH), jnp.float32),
            pltpu.VMEM((_GRP, _CH), jnp.float32),
            pltpu.VMEM((_RPT,), jnp.float32),
            pltpu.VMEM((_NPAD,), jnp.float32),
            pltpu.VMEM_SHARED((_NPAD,), jnp.float32),
            pltpu.SemaphoreType.DMA,
        ],
        compiler_params=pltpu.CompilerParams(needs_layout_passes=False),
    )
    return f(row_p, col_p, ew_p)


def _combine_body(p_ref, h_ref, h0_ref, dis_ref, w_ref, g_ref, be_ref, out_ref):
    inv_deg = dis_ref[...] * dis_ref[...]
    agg = p_ref[0, :_N, :] + p_ref[1, :_N, :] + h_ref[...] * inv_deg
    hcomb = (1.0 - _ALPHA) * agg + _ALPHA * h0_ref[...]
    hc = jnp.dot(hcomb, w_ref[...], preferred_element_type=jnp.float32)
    hc = jnp.maximum(hc, 0.0)
    m = jnp.mean(hc, axis=0, keepdims=True)
    v = jnp.mean((hc - m) ** 2, axis=0, keepdims=True)
    out_ref[...] = g_ref[...] * (hc - m) / jnp.sqrt(v + 1e-5) + be_ref[...]


def _combine(p, h, h0, dis, w, g, be):
    return pl.pallas_call(
        _combine_body,
        out_shape=jax.ShapeDtypeStruct((_N, _H), jnp.float32),
    )(p, h, h0, dis.reshape(_N, 1), w, g.reshape(1, _H), be.reshape(1, _H))


def _bn(x, g, b, eps=1e-5):
    m = jnp.mean(x, axis=0)
    v = jnp.var(x, axis=0)
    return g * (x - m) / jnp.sqrt(v + eps) + b


def _mlp_block(x, W, b, g, be):
    x1 = x @ W + b
    x2 = jax.nn.relu(x1)
    x2 = _bn(x2, g, be)
    return x2 + x1


def kernel(x, edge_attr, x_10d, lin_first, gcn_params, ewmlp_params, head_params, edge_index, batch):
    ew = edge_attr
    for (W, b, g, be) in ewmlp_params:
        ew = _mlp_block(ew, W, b, g, be)
    ew = jax.nn.sigmoid(ew)[:, 0]
    Wf, bf = lin_first
    h = x @ Wf + bf
    h0 = h
    pad = ((0, 0), (0, _EPT - 640000 // _NSHARD))
    row_p = jnp.pad(edge_index[0].reshape(_NSHARD, -1), pad).reshape(_NSHARD, _NCHUNK, _CH)
    col_p = jnp.pad(edge_index[1].reshape(_NSHARD, -1), pad).reshape(_NSHARD, _NCHUNK, _CH)
    ew_p = jnp.pad(ew.reshape(_NSHARD, -1), pad).reshape(_NSHARD, _NCHUNK, _CH)
    norm_p, dis_pad = _norm_sc(row_p, col_p, ew_p)
    dis = dis_pad[:_N]
    for (W1, g, be) in gcn_params:
        p = _spmm(h, row_p, col_p, norm_p)
        h = _combine(p, h, h0, dis, W1, g, be)
    x_aggr = jax.ops.segment_sum(h, batch, num_segments=_NG)
    x_aggr = jax.nn.sigmoid(x_aggr)
    x_aggr = jnp.concatenate([x_aggr, x_10d], axis=1)
    out = x_aggr
    for (W, b, g, be) in head_params:
        out = _mlp_block(out, W, b, g, be)
    out = jax.nn.sigmoid(out)
    return (out, x_aggr)


# speedup vs baseline: 1.0192x; 1.0192x over previous
import functools

import jax
import jax.numpy as jnp
from jax import lax
from jax.experimental import pallas as pl
from jax.experimental.pallas import tpu as pltpu
from jax.experimental.pallas import tpu_sc as plsc

_ALPHA = 0.1
_NG = 64
_N = 10000
_H = 128
_CH = 128
_NCHUNK = 160
_GRP = 16
_NGRP = _NCHUNK // _GRP
_EPT = _CH * _NCHUNK
_NSHARD = 32
_NPAD = 10240
_RPT = _NPAD // 16


def _scale_chunk(buf, normv, i):
    def _escale(eb, c2):
        e0 = eb * 16
        n16 = normv[i, pl.ds(e0, 16)]
        for j in range(16):
            sc = n16[j]
            for k in range(_H // 16):
                sl = pl.ds(k * 16, 16)
                buf[e0 + j, sl] = buf[e0 + j, sl] * sc
        return c2
    lax.fori_loop(0, _CH // 16, _escale, 0)


def _spmm_kernel(h_hbm, row_hbm, col_hbm, norm_hbm, out_hbm,
                 rowv, colv, normv, bufa, bufb, acc, sema, semb, ssa, ssb):
    c = lax.axis_index("c")
    s = lax.axis_index("s")
    wid = c * 16 + s
    def _zrow(i, carry):
        for j in range(_H // 16):
            bufa[i, pl.ds(j * 16, 16)] = jnp.zeros((16,), jnp.float32)
        return carry
    lax.fori_loop(0, _CH, _zrow, 0)
    base = s * _RPT
    for t in range(_RPT // _CH):
        pltpu.sync_copy(bufa, acc.at[pl.ds(base + t * _CH, _CH)])
    plsc.subcore_barrier()

    def _group(g, carry):
        pltpu.sync_copy(row_hbm.at[wid, pl.ds(g * _GRP, _GRP)], rowv)
        pltpu.sync_copy(col_hbm.at[wid, pl.ds(g * _GRP, _GRP)], colv)
        pltpu.sync_copy(norm_hbm.at[wid, pl.ds(g * _GRP, _GRP)], normv)
        pltpu.async_copy(h_hbm.at[rowv.at[0]], bufa, sema)

        def _pair(k, c1):
            i0 = 2 * k
            i1 = i0 + 1
            pltpu.async_copy(h_hbm.at[rowv.at[i1]], bufb, semb)
            pltpu.make_async_copy(h_hbm.at[rowv.at[i0]], bufa, sema).wait()
            _scale_chunk(bufa, normv, i0)
            pltpu.sync_copy(bufa, acc.at[colv.at[i0]], add=True)
            @pl.when(k < _GRP // 2 - 1)
            def _():
                pltpu.async_copy(h_hbm.at[rowv.at[i0 + 2]], bufa, sema)
            pltpu.make_async_copy(h_hbm.at[rowv.at[i1]], bufb, semb).wait()
            _scale_chunk(bufb, normv, i1)
            pltpu.sync_copy(bufb, acc.at[colv.at[i1]], add=True)
            return c1
        lax.fori_loop(0, _GRP // 2, _pair, 0)
        return carry
    lax.fori_loop(0, _NGRP, _group, 0)
    plsc.subcore_barrier()
    for t in range(_RPT // _CH):
        pltpu.sync_copy(acc.at[pl.ds(base + t * _CH, _CH)],
                        out_hbm.at[c, pl.ds(base + t * _CH, _CH)])


def _spmm(h, row_p, col_p, norm_p):
    mesh = plsc.VectorSubcoreMesh(core_axis_name="c", subcore_axis_name="s")
    f = pl.kernel(
        _spmm_kernel,
        out_type=jax.ShapeDtypeStruct((2, _NPAD, _H), jnp.float32),
        mesh=mesh,
        scratch_types=[
            pltpu.VMEM((_GRP, _CH), jnp.int32),
            pltpu.VMEM((_GRP, _CH), jnp.int32),
            pltpu.VMEM((_GRP, _CH), jnp.float32),
            pltpu.VMEM((_CH, _H), jnp.float32),
            pltpu.VMEM((_CH, _H), jnp.float32),
            pltpu.VMEM_SHARED((_NPAD, _H), jnp.float32),
            pltpu.SemaphoreType.DMA,
            pltpu.SemaphoreType.DMA,
            pltpu.SemaphoreType.DMA,
            pltpu.SemaphoreType.DMA,
        ],
        compiler_params=pltpu.CompilerParams(needs_layout_passes=False),
    )
    return f(h, row_p, col_p, norm_p)


def _norm_kernel(row_hbm, col_hbm, ew_hbm, norm_hbm, dis_hbm,
                 rowv, colv, ewv, normb, zbuf, disv, degacc, gsem):
    c = lax.axis_index("c")
    s = lax.axis_index("s")
    wid = c * 16 + s
    def _z(i, carry):
        zbuf[pl.ds(i * 16, 16)] = jnp.zeros((16,), jnp.float32)
        return carry
    lax.fori_loop(0, _RPT // 16, _z, 0)
    pltpu.sync_copy(zbuf, degacc.at[pl.ds(s * _RPT, _RPT)])
    plsc.subcore_barrier()

    def _dgroup(g, carry):
        sh = s + 16 * (g % 2)
        gg = g // 2
        pltpu.sync_copy(col_hbm.at[sh, pl.ds(gg * _GRP, _GRP)], colv)
        pltpu.sync_copy(ew_hbm.at[sh, pl.ds(gg * _GRP, _GRP)], ewv)
        def _dchunk(i, c1):
            pltpu.sync_copy(ewv.at[i], degacc.at[colv.at[i]], add=True)
            return c1
        lax.fori_loop(0, _GRP, _dchunk, 0)
        return carry
    lax.fori_loop(0, 2 * _NGRP, _dgroup, 0)
    plsc.subcore_barrier()

    pltpu.sync_copy(degacc, disv)
    def _rsq(k, carry):
        sl = pl.ds(k * 16, 16)
        xv = disv[sl] + 1.0
        iv = 0x5F3759DF - lax.shift_right_logical(plsc.bitcast(xv, jnp.int32), 1)
        y = plsc.bitcast(iv, jnp.float32)
        xh = 0.5 * xv
        for _ in range(3):
            y = y * (1.5 - xh * y * y)
        disv[sl] = y
        return carry
    lax.fori_loop(0, _NPAD // 16, _rsq, 0)

    def _ngroup(g, carry):
        pltpu.sync_copy(row_hbm.at[wid, pl.ds(g * _GRP, _GRP)], rowv)
        pltpu.sync_copy(col_hbm.at[wid, pl.ds(g * _GRP, _GRP)], colv)
        pltpu.sync_copy(ew_hbm.at[wid, pl.ds(g * _GRP, _GRP)], ewv)
        def _nchunk(i, c1):
            for eb in range(_CH // 16):
                sl = pl.ds(eb * 16, 16)
                r16 = rowv[i, sl]
                c16 = colv[i, sl]
                e16 = ewv[i, sl]
                dr = plsc.load_gather(disv, [r16])
                dc = plsc.load_gather(disv, [c16])
                normb[i, sl] = dr * e16 * dc
            return c1
        lax.fori_loop(0, _GRP, _nchunk, 0)
        pltpu.sync_copy(normb, norm_hbm.at[wid, pl.ds(g * _GRP, _GRP)])
        return carry
    lax.fori_loop(0, _NGRP, _ngroup, 0)
    @pl.when(jnp.logical_and(c == 0, s == 0))
    def _():
        pltpu.sync_copy(disv, dis_hbm)


def _norm_sc(row_p, col_p, ew_p):
    mesh = plsc.VectorSubcoreMesh(core_axis_name="c", subcore_axis_name="s")
    f = pl.kernel(
        _norm_kernel,
        out_type=(jax.ShapeDtypeStruct((_NSHARD, _NCHUNK, _CH), jnp.float32),
                  jax.ShapeDtypeStruct((_NPAD,), jnp.float32)),
        mesh=mesh,
        scratch_types=[
            pltpu.VMEM((_GRP, _CH), jnp.int32),
            pltpu.VMEM((_GRP, _CH), jnp.int32),
            pltpu.VMEM((_GRP, _CH), jnp.float32),
            pltpu.VMEM((_GRP, _CH), jnp.float32),
            pltpu.VMEM((_RPT,), jnp.float32),
            pltpu.VMEM((_NPAD,), jnp.float32),
            pltpu.VMEM_SHARED((_NPAD,), jnp.float32),
            pltpu.SemaphoreType.DMA,
        ],
        compiler_params=pltpu.CompilerParams(needs_layout_passes=False),
    )
    return f(row_p, col_p, ew_p)


def _combine_body(p_ref, h_ref, h0_ref, dis_ref, w_ref, g_ref, be_ref, out_ref):
    inv_deg = dis_ref[...] * dis_ref[...]
    agg = p_ref[0, :_N, :] + p_ref[1, :_N, :] + h_ref[...] * inv_deg
    hcomb = (1.0 - _ALPHA) * agg + _ALPHA * h0_ref[...]
    hc = jnp.dot(hcomb, w_ref[...], preferred_element_type=jnp.float32)
    hc = jnp.maximum(hc, 0.0)
    m = jnp.mean(hc, axis=0, keepdims=True)
    v = jnp.mean((hc - m) ** 2, axis=0, keepdims=True)
    out_ref[...] = g_ref[...] * (hc - m) / jnp.sqrt(v + 1e-5) + be_ref[...]


def _combine(p, h, h0, dis, w, g, be):
    return pl.pallas_call(
        _combine_body,
        out_shape=jax.ShapeDtypeStruct((_N, _H), jnp.float32),
    )(p, h, h0, dis.reshape(_N, 1), w, g.reshape(1, _H), be.reshape(1, _H))


_E = 640000
_EWC = 32000
_NEWC = _E // _EWC
_EW_DIMS = [(5, 4), (4, 3), (3, 2), (2, 1), (1, 1)]


def _ew_body(attr_ref, pw_ref, pb_ref, out_ref, s0, s1, st):
    p = pl.program_id(0)
    cix = pl.program_id(1)
    scratch = (s0, s1)

    @pl.when(jnp.logical_and(p == 0, cix == 0))
    def _init():
        for j in range(8):
            st[0, j] = 0.0
            st[1, j] = 0.0

    @pl.when(jnp.logical_and(p >= 1, cix == 0))
    def _freeze():
        for j in range(8):
            m = st[0, j] / _E
            st[2, j] = m
            st[3, j] = st[1, j] / _E - m * m
            st[0, j] = 0.0
            st[1, j] = 0.0

    sl = pl.ds(cix * _EWC, _EWC)

    def _wv(l, i, j):
        return pw_ref[l * 64 + i * 8 + j]

    def _pv(l, r, j):
        return pb_ref[l * 24 + r * 8 + j]

    def _block_out(l, x1_rows):
        outs = []
        for j in range(len(x1_rows)):
            x1 = x1_rows[j]
            xr = jnp.maximum(x1, 0.0)
            m = st[2, j]
            v = st[3, j]
            x2 = _pv(l, 1, j) * (xr - m) / jnp.sqrt(v + 1e-5) + _pv(l, 2, j)
            outs.append(x2 + x1)
        return outs

    def _matmul_stats(l, xin):
        din, dout = _EW_DIMS[l]
        dst = scratch[l % 2]
        for j in range(dout):
            x1 = _wv(l, 0, j) * xin[0]
            for i in range(1, din):
                x1 = x1 + _wv(l, i, j) * xin[i]
            x1 = x1 + _pv(l, 0, j)
            dst[j, sl] = x1
            xr = jnp.maximum(x1, 0.0)
            st[0, j] += jnp.sum(xr)
            st[1, j] += jnp.sum(xr * xr)

    for ph in range(5):
        @pl.when(p == ph)
        def _(ph=ph):
            if ph == 0:
                xin = [attr_ref[i, :] for i in range(5)]
            else:
                din = _EW_DIMS[ph][0]
                src_s = scratch[(ph - 1) % 2]
                x1_rows = [src_s[i, sl] for i in range(din)]
                xin = _block_out(ph - 1, x1_rows)
            _matmul_stats(ph, xin)
            out_ref[0, :] = jnp.zeros((_EWC,), jnp.float32)

    @pl.when(p == 5)
    def _():
        x1_rows = [scratch[0][0, sl]]
        o = _block_out(4, x1_rows)[0]
        out_ref[0, :] = 1.0 / (1.0 + jnp.exp(-o))


def _edge_mlp(edge_attr, ewmlp_params):
    attr_t = edge_attr.T
    pw = jnp.zeros((5 * 64,), jnp.float32)
    pb = jnp.zeros((5 * 24,), jnp.float32)
    for l, (W, b, g, be) in enumerate(ewmlp_params):
        din, dout = _EW_DIMS[l]
        pw = pw.at[l * 64:(l + 1) * 64].set(
            jnp.pad(W, ((0, 8 - din), (0, 8 - dout))).reshape(-1))
        blk = jnp.stack([jnp.pad(b, (0, 8 - dout)),
                         jnp.pad(g, (0, 8 - dout)),
                         jnp.pad(be, (0, 8 - dout))]).reshape(-1)
        pb = pb.at[l * 24:(l + 1) * 24].set(blk)
    ew = pl.pallas_call(
        _ew_body,
        grid=(6, _NEWC),
        in_specs=[
            pl.BlockSpec((5, _EWC), lambda p, c: (0, c)),
            pl.BlockSpec(memory_space=pltpu.SMEM),
            pl.BlockSpec(memory_space=pltpu.SMEM),
        ],
        out_specs=pl.BlockSpec((1, _EWC), lambda p, c: (0, c)),
        out_shape=jax.ShapeDtypeStruct((1, _E), jnp.float32),
        scratch_shapes=[
            pltpu.VMEM((4, _E), jnp.float32),
            pltpu.VMEM((4, _E), jnp.float32),
            pltpu.SMEM((4, 8), jnp.float32),
        ],
    )(attr_t, pw, pb)
    return ew.reshape(_E)


def _lin_body(x_ref, wf_ref, bf_ref, o_ref):
    o_ref[...] = jnp.dot(x_ref[...], wf_ref[...],
                         preferred_element_type=jnp.float32) + bf_ref[...]


def _first_lin(x, Wf, bf):
    return pl.pallas_call(
        _lin_body,
        out_shape=jax.ShapeDtypeStruct((_N, _H), jnp.float32),
    )(x, Wf, bf.reshape(1, _H))


def _pool_head_body(h_ref, batch_ref, x10_ref,
                    w0, b0, g0, e0, w1, b1, g1, e1, w2, b2, g2, e2,
                    out_ref, xa_ref):
    oh = (batch_ref[...] == lax.broadcasted_iota(jnp.int32, (_N, _NG), 1))
    oh = oh.astype(jnp.float32)
    aggr = lax.dot_general(oh, h_ref[...], (((0,), (0,)), ((), ())),
                           preferred_element_type=jnp.float32)
    xa = 1.0 / (1.0 + jnp.exp(-aggr))
    x10 = x10_ref[...]
    xa_ref[:, :_H] = xa
    xa_ref[:, _H:] = x10
    z = jnp.concatenate([xa, x10], axis=1)
    for (w, b, g, e) in ((w0, b0, g0, e0), (w1, b1, g1, e1), (w2, b2, g2, e2)):
        x1 = jnp.dot(z, w[...], preferred_element_type=jnp.float32) + b[...]
        xr = jnp.maximum(x1, 0.0)
        m = jnp.mean(xr, axis=0, keepdims=True)
        v = jnp.mean((xr - m) ** 2, axis=0, keepdims=True)
        z = g[...] * (xr - m) / jnp.sqrt(v + 1e-5) + e[...] + x1
    out_ref[...] = 1.0 / (1.0 + jnp.exp(-z))


def _pool_head(h, batch, x_10d, head_params):
    args = [h, batch.reshape(_N, 1), x_10d]
    for (W, b, g, be) in head_params:
        d = W.shape[1]
        args += [W, b.reshape(1, d), g.reshape(1, d), be.reshape(1, d)]
    return pl.pallas_call(
        _pool_head_body,
        out_shape=(jax.ShapeDtypeStruct((_NG, 4), jnp.float32),
                   jax.ShapeDtypeStruct((_NG, _H + 10), jnp.float32)),
    )(*args)


def kernel(x, edge_attr, x_10d, lin_first, gcn_params, ewmlp_params, head_params, edge_index, batch):
    ew = _edge_mlp(edge_attr, ewmlp_params)
    Wf, bf = lin_first
    h = _first_lin(x, Wf, bf)
    h0 = h
    pad = ((0, 0), (0, _EPT - _E // _NSHARD))
    row_p = jnp.pad(edge_index[0].reshape(_NSHARD, -1), pad).reshape(_NSHARD, _NCHUNK, _CH)
    col_p = jnp.pad(edge_index[1].reshape(_NSHARD, -1), pad).reshape(_NSHARD, _NCHUNK, _CH)
    ew_p = jnp.pad(ew.reshape(_NSHARD, -1), pad).reshape(_NSHARD, _NCHUNK, _CH)
    norm_p, dis_pad = _norm_sc(row_p, col_p, ew_p)
    dis = dis_pad[:_N]
    for (W1, g, be) in gcn_params:
        p = _spmm(h, row_p, col_p, norm_p)
        h = _combine(p, h, h0, dis, W1, g, be)
    out, x_aggr = _pool_head(h, batch, x_10d, head_params)
    return (out, x_aggr)
